# Initial kernel scaffold; baseline (speedup 1.0000x reference)
import jax
import jax.numpy as jnp
from jax.experimental import pallas as pl


def kernel(x, edges, W1, b1, W2, b2, W3, b3):
    raise NotImplementedError("write your pallas kernel here")


# speedup vs baseline: 8.4075x; 8.4075x over previous
import functools

import jax
import jax.numpy as jnp
from jax import lax
from jax.experimental import pallas as pl
from jax.experimental.pallas import tpu as pltpu
from jax.experimental.pallas import tpu_sc as plsc

N = 10000
NP = 10240
NE = 160000
F = 128
TILES = 32
CW = 128
CHUNKS = (NE + TILES * CW - 1) // (TILES * CW)
NE_PAD = TILES * CHUNKS * CW
ROWS_PER_TILE = NP // 16
RB = 1024
GRID = NP // RB

_MESH = plsc.VectorSubcoreMesh(core_axis_name="c", subcore_axis_name="s")


def _sc_deg(dst2d, ones_hbm_v, zeros_hbm_v):
    @functools.partial(
        pl.kernel,
        out_type=jax.ShapeDtypeStruct((2, NP, F), jnp.float32),
        mesh=_MESH,
        scratch_types=[
            pltpu.VMEM((CHUNKS, CW), jnp.int32),
            pltpu.VMEM((CW, F), jnp.float32),
            pltpu.VMEM_SHARED((NP, F), jnp.float32),
        ],
    )
    def k(dst_hbm, ones_hbm, zeros_hbm, out_hbm, dst_v, ones_v, acc):
        c = lax.axis_index("c")
        s = lax.axis_index("s")
        wid = c * 16 + s
        row0 = s * ROWS_PER_TILE
        pltpu.sync_copy(zeros_hbm, acc.at[pl.ds(row0, ROWS_PER_TILE)])
        pltpu.sync_copy(ones_hbm, ones_v)
        pltpu.sync_copy(dst_hbm.at[pl.ds(wid * CHUNKS, CHUNKS)], dst_v)
        plsc.subcore_barrier()

        def body(j, _):
            pltpu.sync_copy(ones_v, acc.at[dst_v.at[j]], add=True)
            return ()

        lax.fori_loop(0, CHUNKS, body, ())
        plsc.subcore_barrier()
        pltpu.sync_copy(acc.at[pl.ds(row0, ROWS_PER_TILE)],
                        out_hbm.at[c, pl.ds(row0, ROWS_PER_TILE)])

    return k(dst2d, ones_hbm_v, zeros_hbm_v)


def _sc_prop(y, src2d, dst2d, zeros_hbm_v):
    @functools.partial(
        pl.kernel,
        out_type=jax.ShapeDtypeStruct((2, NP, F), jnp.float32),
        mesh=_MESH,
        scratch_types=[
            pltpu.VMEM((CHUNKS, CW), jnp.int32),
            pltpu.VMEM((CHUNKS, CW), jnp.int32),
            pltpu.VMEM((CW, F), jnp.float32),
            pltpu.VMEM_SHARED((NP, F), jnp.float32),
            pltpu.SemaphoreType.DMA,
        ],
    )
    def k(y_hbm, src_hbm, dst_hbm, zeros_hbm, out_hbm,
          src_v, dst_v, rows_v, acc, sem):
        c = lax.axis_index("c")
        s = lax.axis_index("s")
        wid = c * 16 + s
        row0 = s * ROWS_PER_TILE
        pltpu.sync_copy(zeros_hbm, acc.at[pl.ds(row0, ROWS_PER_TILE)])
        pltpu.sync_copy(src_hbm.at[pl.ds(wid * CHUNKS, CHUNKS)], src_v)
        pltpu.sync_copy(dst_hbm.at[pl.ds(wid * CHUNKS, CHUNKS)], dst_v)
        plsc.subcore_barrier()

        def body(j, _):
            pltpu.async_copy(y_hbm.at[src_v.at[j]], rows_v, sem).wait()
            pltpu.sync_copy(rows_v, acc.at[dst_v.at[j]], add=True)
            return ()

        lax.fori_loop(0, CHUNKS, body, ())
        plsc.subcore_barrier()
        pltpu.sync_copy(acc.at[pl.ds(row0, ROWS_PER_TILE)],
                        out_hbm.at[c, pl.ds(row0, ROWS_PER_TILE)])

    return k(y, src2d, dst2d, zeros_hbm_v)


def _pad128(v):
    return jnp.concatenate([v, jnp.zeros_like(v)], axis=1)


def _tc_a(deg2, x, W1):
    def body(deg_ref, x_ref, w_ref, dinv_ref, y_ref):
        d = deg_ref[0, :, 0:1] + deg_ref[1, :, 0:1] + 1.0
        dinv = lax.rsqrt(d)
        dinv_ref[...] = dinv
        y_ref[...] = _pad128(dinv * jnp.dot(x_ref[...], w_ref[...],
                                            preferred_element_type=jnp.float32))

    return pl.pallas_call(
        body,
        grid=(GRID,),
        in_specs=[
            pl.BlockSpec((2, RB, F), lambda i: (0, i, 0)),
            pl.BlockSpec((RB, 640), lambda i: (i, 0)),
            pl.BlockSpec((640, 64), lambda i: (0, 0)),
        ],
        out_specs=[
            pl.BlockSpec((RB, 1), lambda i: (i, 0)),
            pl.BlockSpec((RB, F), lambda i: (i, 0)),
        ],
        out_shape=[
            jax.ShapeDtypeStruct((NP, 1), jnp.float32),
            jax.ShapeDtypeStruct((NP, F), jnp.float32),
        ],
    )(deg2, x, W1)


def _tc_mid(p, y, dinv, b, W):
    def body(p_ref, y_ref, dinv_ref, b_ref, w_ref, o_ref):
        dinv = dinv_ref[...]
        agg = p_ref[0, :, 0:64] + p_ref[1, :, 0:64] + y_ref[:, 0:64]
        h = jnp.maximum(dinv * agg + b_ref[...], 0.0)
        o_ref[...] = _pad128(dinv * jnp.dot(h, w_ref[...],
                                            preferred_element_type=jnp.float32))

    return pl.pallas_call(
        body,
        grid=(GRID,),
        in_specs=[
            pl.BlockSpec((2, RB, F), lambda i: (0, i, 0)),
            pl.BlockSpec((RB, F), lambda i: (i, 0)),
            pl.BlockSpec((RB, 1), lambda i: (i, 0)),
            pl.BlockSpec((1, 64), lambda i: (0, 0)),
            pl.BlockSpec((64, 64), lambda i: (0, 0)),
        ],
        out_specs=pl.BlockSpec((RB, F), lambda i: (i, 0)),
        out_shape=jax.ShapeDtypeStruct((NP, F), jnp.float32),
    )(p, y, dinv, b, W)


def _tc_c(p, y, dinv, b):
    def body(p_ref, y_ref, dinv_ref, b_ref, o_ref):
        dinv = dinv_ref[...]
        agg = p_ref[0, :, 0:64] + p_ref[1, :, 0:64] + y_ref[:, 0:64]
        h = jnp.maximum(dinv * agg + b_ref[...], 0.0)
        o_ref[...] = _pad128(dinv * h)

    return pl.pallas_call(
        body,
        grid=(GRID,),
        in_specs=[
            pl.BlockSpec((2, RB, F), lambda i: (0, i, 0)),
            pl.BlockSpec((RB, F), lambda i: (i, 0)),
            pl.BlockSpec((RB, 1), lambda i: (i, 0)),
            pl.BlockSpec((1, 64), lambda i: (0, 0)),
        ],
        out_specs=pl.BlockSpec((RB, F), lambda i: (i, 0)),
        out_shape=jax.ShapeDtypeStruct((NP, F), jnp.float32),
    )(p, y, dinv, b)


def _tc_d(p, y, dinv, W3, b3):
    def body(p_ref, y_ref, dinv_ref, w_ref, b_ref, o_ref):
        agg = dinv_ref[...] * (p_ref[0, :, 0:64] + p_ref[1, :, 0:64]
                               + y_ref[:, 0:64])
        o_ref[...] = jnp.dot(agg, w_ref[...],
                             preferred_element_type=jnp.float32) + b_ref[...]

    return pl.pallas_call(
        body,
        grid=(GRID,),
        in_specs=[
            pl.BlockSpec((2, RB, F), lambda i: (0, i, 0)),
            pl.BlockSpec((RB, F), lambda i: (i, 0)),
            pl.BlockSpec((RB, 1), lambda i: (i, 0)),
            pl.BlockSpec((64, 640), lambda i: (0, 0)),
            pl.BlockSpec((1, 640), lambda i: (0, 0)),
        ],
        out_specs=pl.BlockSpec((RB, 640), lambda i: (i, 0)),
        out_shape=jax.ShapeDtypeStruct((NP, 640), jnp.float32),
    )(p, y, dinv, W3, b3)


def kernel(x, edges, W1, b1, W2, b2, W3, b3):
    src = edges[:, 0]
    dst = edges[:, 1]
    pad = NE_PAD - NE
    src2d = jnp.concatenate(
        [src, jnp.zeros((pad,), jnp.int32)]).reshape(TILES * CHUNKS, CW)
    dst2d = jnp.concatenate(
        [dst, jnp.full((pad,), NP - 1, jnp.int32)]).reshape(TILES * CHUNKS, CW)
    x_pad = jnp.pad(x, ((0, NP - N), (0, 0)))

    ones_v = jnp.ones((CW, F), jnp.float32)
    zeros_v = jnp.zeros((ROWS_PER_TILE, F), jnp.float32)

    deg2 = _sc_deg(dst2d, ones_v, zeros_v)
    dinv, y1 = _tc_a(deg2, x_pad, W1)
    p1 = _sc_prop(y1, src2d, dst2d, zeros_v)
    y2 = _tc_mid(p1, y1, dinv, b1.reshape(1, 64), W2)
    p2 = _sc_prop(y2, src2d, dst2d, zeros_v)
    y3 = _tc_c(p2, y2, dinv, b2.reshape(1, 64))
    p3 = _sc_prop(y3, src2d, dst2d, zeros_v)
    out = _tc_d(p3, y3, dinv, W3, b3.reshape(1, 640))
    return out[:N]


# speedup vs baseline: 18.9717x; 2.2565x over previous
import functools

import jax
import jax.numpy as jnp
from jax import lax
from jax.experimental import pallas as pl
from jax.experimental.pallas import tpu as pltpu
from jax.experimental.pallas import tpu_sc as plsc

N = 10000
NP = 10240
NE = 160000
F = 128
TILES = 32
CW = 128
CHUNKS = (NE + TILES * CW - 1) // (TILES * CW)
NE_PAD = TILES * CHUNKS * CW
ROWS_PER_TILE = NP // 16
RB = 1024
GRID = NP // RB

_MESH = plsc.VectorSubcoreMesh(core_axis_name="c", subcore_axis_name="s")


def _sc_deg(dst2d, ones_hbm_v, zeros_hbm_v):
    @functools.partial(
        pl.kernel,
        out_type=jax.ShapeDtypeStruct((2, NP, F), jnp.float32),
        mesh=_MESH,
        scratch_types=[
            pltpu.VMEM((CHUNKS, CW), jnp.int32),
            pltpu.VMEM((CW, F), jnp.float32),
            pltpu.VMEM_SHARED((NP, F), jnp.float32),
        ],
    )
    def k(dst_hbm, ones_hbm, zeros_hbm, out_hbm, dst_v, ones_v, acc):
        c = lax.axis_index("c")
        s = lax.axis_index("s")
        wid = c * 16 + s
        row0 = s * ROWS_PER_TILE
        pltpu.sync_copy(zeros_hbm, acc.at[pl.ds(row0, ROWS_PER_TILE)])
        pltpu.sync_copy(ones_hbm, ones_v)
        pltpu.sync_copy(dst_hbm.at[pl.ds(wid * CHUNKS, CHUNKS)], dst_v)
        plsc.subcore_barrier()

        def body(j, _):
            pltpu.sync_copy(ones_v, acc.at[dst_v.at[j]], add=True)
            return ()

        lax.fori_loop(0, CHUNKS, body, ())
        plsc.subcore_barrier()
        pltpu.sync_copy(acc.at[pl.ds(row0, ROWS_PER_TILE)],
                        out_hbm.at[c, pl.ds(row0, ROWS_PER_TILE)])

    return k(dst2d, ones_hbm_v, zeros_hbm_v)


def _sc_prop(y, src2d, dst2d, zeros_hbm_v):
    @functools.partial(
        pl.kernel,
        out_type=jax.ShapeDtypeStruct((2, NP, F), jnp.float32),
        mesh=_MESH,
        scratch_types=[
            pltpu.VMEM((CHUNKS, CW), jnp.int32),
            pltpu.VMEM((CHUNKS, CW), jnp.int32),
            pltpu.VMEM((CW, F), jnp.float32),
            pltpu.VMEM_SHARED((NP, F), jnp.float32),
            pltpu.SemaphoreType.DMA,
        ],
    )
    def k(y_hbm, src_hbm, dst_hbm, zeros_hbm, out_hbm,
          src_v, dst_v, rows_v, acc, sem):
        c = lax.axis_index("c")
        s = lax.axis_index("s")
        wid = c * 16 + s
        row0 = s * ROWS_PER_TILE
        pltpu.sync_copy(zeros_hbm, acc.at[pl.ds(row0, ROWS_PER_TILE)])
        pltpu.sync_copy(src_hbm.at[pl.ds(wid * CHUNKS, CHUNKS)], src_v)
        pltpu.sync_copy(dst_hbm.at[pl.ds(wid * CHUNKS, CHUNKS)], dst_v)
        plsc.subcore_barrier()

        def body(j, _):
            pltpu.async_copy(y_hbm.at[src_v.at[j]], rows_v, sem).wait()
            pltpu.sync_copy(rows_v, acc.at[dst_v.at[j]], add=True)
            return ()

        lax.fori_loop(0, CHUNKS, body, ())
        plsc.subcore_barrier()
        pltpu.sync_copy(acc.at[pl.ds(row0, ROWS_PER_TILE)],
                        out_hbm.at[c, pl.ds(row0, ROWS_PER_TILE)])

    return k(y, src2d, dst2d, zeros_hbm_v)


def _pad128(v):
    return jnp.concatenate([v, jnp.zeros_like(v)], axis=1)


def _tc_a(deg2, x, W1):
    def body(deg_ref, x_ref, w_ref, dinv_ref, y_ref):
        d = deg_ref[0, :, 0:1] + deg_ref[1, :, 0:1] + 1.0
        dinv = lax.rsqrt(d)
        dinv_ref[...] = dinv
        y_ref[...] = _pad128(dinv * jnp.dot(x_ref[...], w_ref[...],
                                            preferred_element_type=jnp.float32))

    return pl.pallas_call(
        body,
        grid=(GRID,),
        in_specs=[
            pl.BlockSpec((2, RB, F), lambda i: (0, i, 0)),
            pl.BlockSpec((RB, 640), lambda i: (i, 0)),
            pl.BlockSpec((640, 64), lambda i: (0, 0)),
        ],
        out_specs=[
            pl.BlockSpec((RB, 1), lambda i: (i, 0)),
            pl.BlockSpec((RB, F), lambda i: (i, 0)),
        ],
        out_shape=[
            jax.ShapeDtypeStruct((NP, 1), jnp.float32),
            jax.ShapeDtypeStruct((NP, F), jnp.float32),
        ],
    )(deg2, x, W1)


def _tc_mid(p, y, dinv, b, W):
    def body(p_ref, y_ref, dinv_ref, b_ref, w_ref, o_ref):
        dinv = dinv_ref[...]
        agg = p_ref[0, :, 0:64] + p_ref[1, :, 0:64] + y_ref[:, 0:64]
        h = jnp.maximum(dinv * agg + b_ref[...], 0.0)
        o_ref[...] = _pad128(dinv * jnp.dot(h, w_ref[...],
                                            preferred_element_type=jnp.float32))

    return pl.pallas_call(
        body,
        grid=(GRID,),
        in_specs=[
            pl.BlockSpec((2, RB, F), lambda i: (0, i, 0)),
            pl.BlockSpec((RB, F), lambda i: (i, 0)),
            pl.BlockSpec((RB, 1), lambda i: (i, 0)),
            pl.BlockSpec((1, 64), lambda i: (0, 0)),
            pl.BlockSpec((64, 64), lambda i: (0, 0)),
        ],
        out_specs=pl.BlockSpec((RB, F), lambda i: (i, 0)),
        out_shape=jax.ShapeDtypeStruct((NP, F), jnp.float32),
    )(p, y, dinv, b, W)


def _tc_c(p, y, dinv, b):
    def body(p_ref, y_ref, dinv_ref, b_ref, o_ref):
        dinv = dinv_ref[...]
        agg = p_ref[0, :, 0:64] + p_ref[1, :, 0:64] + y_ref[:, 0:64]
        h = jnp.maximum(dinv * agg + b_ref[...], 0.0)
        o_ref[...] = _pad128(dinv * h)

    return pl.pallas_call(
        body,
        grid=(GRID,),
        in_specs=[
            pl.BlockSpec((2, RB, F), lambda i: (0, i, 0)),
            pl.BlockSpec((RB, F), lambda i: (i, 0)),
            pl.BlockSpec((RB, 1), lambda i: (i, 0)),
            pl.BlockSpec((1, 64), lambda i: (0, 0)),
        ],
        out_specs=pl.BlockSpec((RB, F), lambda i: (i, 0)),
        out_shape=jax.ShapeDtypeStruct((NP, F), jnp.float32),
    )(p, y, dinv, b)


def _tc_d(p, y, dinv, W3, b3):
    def body(p_ref, y_ref, dinv_ref, w_ref, b_ref, o_ref):
        agg = dinv_ref[...] * (p_ref[0, :, 0:64] + p_ref[1, :, 0:64]
                               + y_ref[:, 0:64])
        o_ref[...] = jnp.dot(agg, w_ref[...],
                             preferred_element_type=jnp.float32) + b_ref[...]

    return pl.pallas_call(
        body,
        grid=(GRID,),
        in_specs=[
            pl.BlockSpec((2, RB, F), lambda i: (0, i, 0)),
            pl.BlockSpec((RB, F), lambda i: (i, 0)),
            pl.BlockSpec((RB, 1), lambda i: (i, 0)),
            pl.BlockSpec((64, 640), lambda i: (0, 0)),
            pl.BlockSpec((1, 640), lambda i: (0, 0)),
        ],
        out_specs=pl.BlockSpec((RB, 640), lambda i: (i, 0)),
        out_shape=jax.ShapeDtypeStruct((NP, 640), jnp.float32),
    )(p, y, dinv, W3, b3)


def kernel(x, edges, W1, b1, W2, b2, W3, b3):
    src = edges[:, 0]
    dst = edges[:, 1]
    pad = NE_PAD - NE
    pad_iota = jnp.arange(pad, dtype=jnp.int32)
    src2d = jnp.concatenate(
        [src, pad_iota % N]).reshape(TILES * CHUNKS, CW)
    dst2d = jnp.concatenate(
        [dst, N + pad_iota % (NP - N)]).reshape(TILES * CHUNKS, CW)
    x_pad = jnp.pad(x, ((0, NP - N), (0, 0)))

    ones_v = jnp.ones((CW, F), jnp.float32)
    zeros_v = jnp.zeros((ROWS_PER_TILE, F), jnp.float32)

    deg2 = _sc_deg(dst2d, ones_v, zeros_v)
    dinv, y1 = _tc_a(deg2, x_pad, W1)
    p1 = _sc_prop(y1, src2d, dst2d, zeros_v)
    y2 = _tc_mid(p1, y1, dinv, b1.reshape(1, 64), W2)
    p2 = _sc_prop(y2, src2d, dst2d, zeros_v)
    y3 = _tc_c(p2, y2, dinv, b2.reshape(1, 64))
    p3 = _sc_prop(y3, src2d, dst2d, zeros_v)
    out = _tc_d(p3, y3, dinv, W3, b3.reshape(1, 640))
    return out[:N]
